# Initial kernel scaffold; baseline (speedup 1.0000x reference)
import jax
import jax.numpy as jnp
from jax.experimental import pallas as pl


def kernel(x, w_red, wg, W1, b1, W2, b2):
    raise NotImplementedError("write your pallas kernel here")


# speedup vs baseline: 1.0502x; 1.0502x over previous
import functools

import jax
import jax.numpy as jnp
from jax import lax
from jax.experimental import pallas as pl
from jax.experimental.pallas import tpu as pltpu
from jax.experimental.pallas import tpu_sc as plsc

_B, _T, _C, _H, _E = 1, 2048, 768, 768, 8

_G = 128
_NB = _T // _G + _E
_TPAD = _NB * _G
_NC, _NS = 2, 16
_NW = _NC * _NS
_CHUNK = _TPAD // _NW


def _gate_body(x_ref, wrt_ref, wg_ref, sc_ref, id_ref):
    xg = x_ref[...]
    g = jnp.dot(xg, wrt_ref[...], preferred_element_type=jnp.float32)
    wgv = wg_ref[...]
    nrm = jnp.sqrt(jnp.sum(wgv * wgv, axis=1, keepdims=True))
    wgr = (1.5 / jnp.maximum(nrm, 1e-12)) * wgv
    nrm2 = jnp.sqrt(jnp.sum(wgr * wgr, axis=1, keepdims=True))
    wgn = wgr / jnp.maximum(nrm2, 1e-4)
    logits = lax.dot_general(g, wgn, (((1,), (1,)), ((), ())),
                             preferred_element_type=jnp.float32)
    m = jnp.max(logits, axis=1, keepdims=True)
    p = jnp.exp(logits - m)
    gates = p / jnp.sum(p, axis=1, keepdims=True)
    mx = jnp.max(gates, axis=1, keepdims=True)
    ii = lax.broadcasted_iota(jnp.int32, (_T, _E), 1)
    cand = jnp.where(gates >= mx, ii, _E)
    id_ref[...] = jnp.min(cand, axis=1, keepdims=True)
    sc_ref[...] = mx


def _gate(x2, w_redT, wg):
    return pl.pallas_call(
        _gate_body,
        out_shape=[jax.ShapeDtypeStruct((_T, 1), jnp.float32),
                   jax.ShapeDtypeStruct((_T, 1), jnp.int32)],
    )(x2, w_redT, wg)


@functools.lru_cache(maxsize=1)
def _sc_kernels():
    mesh = plsc.VectorSubcoreMesh(core_axis_name="c", subcore_axis_name="s",
                                  num_cores=_NC, num_subcores=_NS)

    @functools.partial(
        pl.kernel,
        out_type=[jax.ShapeDtypeStruct((_TPAD, _C), jnp.float32),
                  jax.ShapeDtypeStruct((_TPAD, 128), jnp.float32)],
        mesh=mesh,
        scratch_types=[
            pltpu.VMEM((_CHUNK,), jnp.int32),
            pltpu.VMEM((_CHUNK,), jnp.int32),
            pltpu.VMEM((_CHUNK, _C), jnp.float32),
            pltpu.VMEM((_CHUNK, 128), jnp.float32),
            pltpu.SemaphoreType.DMA,
            pltpu.SemaphoreType.DMA,
        ],
    )
    def sc_gather(x_hbm, scx_hbm, gidx_hbm, sgidx_hbm, xs_hbm, ss_hbm,
                  idx_v, sidx_v, rows_v, srows_v, sem, sem2):
        wid = lax.axis_index("s") * _NC + lax.axis_index("c")
        base = wid * _CHUNK
        pltpu.sync_copy(gidx_hbm.at[pl.ds(base, _CHUNK)], idx_v)
        pltpu.sync_copy(sgidx_hbm.at[pl.ds(base, _CHUNK)], sidx_v)
        cp1 = pltpu.async_copy(x_hbm.at[idx_v], rows_v, sem)
        cp2 = pltpu.async_copy(scx_hbm.at[sidx_v], srows_v, sem2)
        cp1.wait()
        cp2.wait()
        pltpu.sync_copy(rows_v, xs_hbm.at[pl.ds(base, _CHUNK)])
        pltpu.sync_copy(srows_v, ss_hbm.at[pl.ds(base, _CHUNK)])

    @functools.partial(
        pl.kernel,
        out_type=jax.ShapeDtypeStruct((_T + 8, _C), jnp.float32),
        mesh=mesh,
        scratch_types=[
            pltpu.VMEM((_CHUNK,), jnp.int32),
            pltpu.VMEM((_CHUNK, _C), jnp.float32),
            pltpu.SemaphoreType.DMA,
        ],
    )
    def sc_scatter(ys_hbm, sctidx_hbm, out_hbm, idx_v, rows_v, sem):
        wid = lax.axis_index("s") * _NC + lax.axis_index("c")
        base = wid * _CHUNK
        pltpu.sync_copy(sctidx_hbm.at[pl.ds(base, _CHUNK)], idx_v)
        pltpu.sync_copy(ys_hbm.at[pl.ds(base, _CHUNK)], rows_v)
        pltpu.async_copy(rows_v, out_hbm.at[idx_v], sem).wait()

    return sc_gather, sc_scatter


def _sc_gather(x2, scx, gidx, sgidx):
    return _sc_kernels()[0](x2, scx, gidx, sgidx)


def _sc_scatter(ys, sctidx):
    return _sc_kernels()[1](ys, sctidx)


def _ffn_body(eid_ref, xs_ref, w1_ref, b1_ref, w2_ref, b2_ref, ss_ref, out_ref):
    del eid_ref
    xb = xs_ref[...]
    h = jnp.dot(xb, w1_ref[0], preferred_element_type=jnp.float32) + b1_ref[0]
    z = h * 0.7071067811865476
    a = jnp.abs(z)
    t = 1.0 / (1.0 + 0.3275911 * a)
    poly = t * (0.254829592 + t * (-0.284496736 + t * (1.421413741
               + t * (-1.453152027 + t * 1.061405429))))
    erf_a = 1.0 - poly * jnp.exp(-a * a)
    erf_z = jnp.where(z < 0.0, -erf_a, erf_a)
    h = 0.5 * h * (1.0 + erf_z)
    y = jnp.dot(h, w2_ref[0], preferred_element_type=jnp.float32) + b2_ref[0]
    out_ref[...] = y * ss_ref[...]


def _ffn(blk_eid, xs, W1, b1r, W2, b2r, ss2):
    grid_spec = pltpu.PrefetchScalarGridSpec(
        num_scalar_prefetch=1,
        grid=(_NB,),
        in_specs=[
            pl.BlockSpec((_G, _C), lambda i, eid: (i, 0)),
            pl.BlockSpec((1, _C, _H), lambda i, eid: (eid[i], 0, 0)),
            pl.BlockSpec((1, 1, _H), lambda i, eid: (eid[i], 0, 0)),
            pl.BlockSpec((1, _H, _C), lambda i, eid: (eid[i], 0, 0)),
            pl.BlockSpec((1, 1, _C), lambda i, eid: (eid[i], 0, 0)),
            pl.BlockSpec((_G, 1), lambda i, eid: (i, 0)),
        ],
        out_specs=pl.BlockSpec((_G, _C), lambda i, eid: (i, 0)),
    )
    return pl.pallas_call(
        _ffn_body,
        grid_spec=grid_spec,
        out_shape=jax.ShapeDtypeStruct((_TPAD, _C), jnp.float32),
        compiler_params=pltpu.CompilerParams(
            dimension_semantics=("arbitrary",)),
    )(blk_eid, xs, W1, b1r, W2, b2r, ss2)


def kernel(x, w_red, wg, W1, b1, W2, b2):
    Bx, Tx, Cx = x.shape
    x2 = x.reshape(Tx, Cx)

    scores2, idx2 = _gate(x2, w_red.T, wg)
    idx = idx2[:, 0]

    i32 = jnp.int32
    oh = (idx[:, None] == jnp.arange(_E, dtype=i32)[None, :]).astype(i32)
    pos = jnp.cumsum(oh, axis=0)
    pos_in = jnp.take_along_axis(pos, idx[:, None], axis=1)[:, 0] - 1
    counts = pos[-1]
    ntiles = (counts + _G - 1) // _G
    cumblk = jnp.cumsum(ntiles)
    pad_start = (jnp.concatenate([jnp.zeros((1,), i32), cumblk[:-1]]) * _G)
    ppos = pad_start[idx] + pos_in
    tok = jnp.arange(_T, dtype=i32)
    gidx = jnp.zeros((_TPAD,), i32).at[ppos].set(tok)
    sgidx = jnp.full((_TPAD,), _T, i32).at[ppos].set(tok)
    trash = _T + (jnp.arange(_TPAD, dtype=i32) % 8)
    sctidx = trash.at[ppos].set(tok)
    blk = jnp.arange(_NB, dtype=i32)
    blk_eid = jnp.minimum(
        jnp.searchsorted(cumblk, blk, side="right").astype(i32), _E - 1)

    scx = jnp.broadcast_to(
        jnp.concatenate([scores2, jnp.zeros((16, 1), jnp.float32)]),
        (_T + 16, 128))

    xs, ss = _sc_gather(x2, scx, gidx, sgidx)
    ys = _ffn(blk_eid, xs, W1, b1.reshape(_E, 1, _H), W2,
              b2.reshape(_E, 1, _C), ss[:, :1])
    out_pad = _sc_scatter(ys, sctidx)

    out = out_pad[:_T].reshape(Bx, Tx, Cx)
    return (out, jnp.sum(out))
